# baseline (speedup 1.0000x reference)
import functools

import jax
import jax.numpy as jnp
from jax import lax
from jax.experimental import pallas as pl
from jax.experimental.pallas import tpu as pltpu
from jax.experimental.pallas import tpu_sc as plsc

_VOCAB = 1000
_EMBED = 64
_B = 4096
_L = 200

_NC = 2
_NS = 16
_NW = _NC * _NS
_BW = _B // _NW
_G = _BW // 16


@functools.partial(
    pl.kernel,
    mesh=plsc.VectorSubcoreMesh(core_axis_name="c", subcore_axis_name="s"),
    out_type=jax.ShapeDtypeStruct((_L, _EMBED, _B), jnp.float32),
    scratch_types=[
        pltpu.VMEM((_L, _BW), jnp.int32),
        pltpu.VMEM((_VOCAB * _EMBED,), jnp.float32),
        pltpu.VMEM((2, _EMBED, _BW), jnp.float32),
        [pltpu.SemaphoreType.DMA] * 2,
    ],
    compiler_params=pltpu.CompilerParams(use_tc_tiling_on_sc=True,
                                         needs_layout_passes=False),
)
def _gather_kernel(idx_hbm, table_hbm, out_hbm, idx_v, table_v, stage_v,
                   ssems):
    wid = lax.axis_index("s") * _NC + lax.axis_index("c")
    b0 = wid * _BW

    pltpu.sync_copy(table_hbm, table_v)
    pltpu.sync_copy(idx_hbm.at[wid], idx_v)

    def fire_store(l, par):
        return pltpu.async_copy(stage_v.at[par],
                                out_hbm.at[l, :, pl.ds(b0, _BW)], ssems[par])

    sd = [pltpu.make_async_copy(stage_v.at[par],
                                out_hbm.at[0, :, pl.ds(b0, _BW)], ssems[par])
          for par in (0, 1)]

    jj = lax.iota(jnp.int32, 16)

    def build_block(l, par):
        @plsc.parallel_loop(0, _G, 1, unroll=8)
        def g_body(g):
            base = idx_v[l, pl.ds(g * 16, 16)] * _EMBED
            bvec = jj + g * 16

            @plsc.parallel_loop(0, _EMBED, 16)
            def e_body(e0):
                base_e0 = base + e0
                for k in range(16):
                    rot = (jj + k) & 15
                    v = plsc.load_gather(table_v, [base_e0 + rot])
                    plsc.store_scatter(stage_v.at[par],
                                       [rot + e0, bvec], v)

    for par in (0, 1):
        build_block(par, par)
        fire_store(par, par)

    def body(j, carry):
        for par in (0, 1):
            l = 2 + j * 2 + par
            sd[par].wait()
            build_block(l, par)
            fire_store(l, par)
        return carry

    lax.fori_loop(0, (_L - 2) // 2, body, 0)

    sd[0].wait()
    sd[1].wait()


def kernel(seqs, att_mask, word_embedding):
    del att_mask
    idx = seqs.T.reshape(_L, _NW, _BW).transpose(1, 0, 2)
    table = word_embedding.reshape(_VOCAB * _EMBED)
    out_t = _gather_kernel(idx, table)
    return out_t.transpose(2, 0, 1)


# speedup vs baseline: 1.2866x; 1.2866x over previous
import functools

import jax
import jax.numpy as jnp
from jax import lax
from jax.experimental import pallas as pl
from jax.experimental.pallas import tpu as pltpu
from jax.experimental.pallas import tpu_sc as plsc

_VOCAB = 1000
_EMBED = 64
_B = 4096
_L = 200

_NC = 2
_NS = 16
_NW = _NC * _NS
_BW = _B // _NW
_G = _BW // 16


@functools.partial(
    pl.kernel,
    mesh=plsc.VectorSubcoreMesh(core_axis_name="c", subcore_axis_name="s"),
    out_type=jax.ShapeDtypeStruct((_L, _EMBED, _B), jnp.float32),
    scratch_types=[
        pltpu.VMEM((_L, _BW), jnp.int32),
        pltpu.VMEM((_VOCAB * _EMBED,), jnp.float32),
        pltpu.VMEM((4, _EMBED, _BW), jnp.float32),
        [pltpu.SemaphoreType.DMA] * 4,
    ],
    compiler_params=pltpu.CompilerParams(use_tc_tiling_on_sc=True,
                                         needs_layout_passes=False),
)
def _gather_kernel(idx_hbm, table_hbm, out_hbm, idx_v, table_v, stage_v,
                   ssems):
    wid = lax.axis_index("s") * _NC + lax.axis_index("c")
    b0 = wid * _BW

    pltpu.sync_copy(table_hbm, table_v)
    pltpu.sync_copy(idx_hbm.at[wid], idx_v)

    def fire_store(l, par):
        return pltpu.async_copy(stage_v.at[par],
                                out_hbm.at[l, :, pl.ds(b0, _BW)], ssems[par])

    sd = [pltpu.make_async_copy(stage_v.at[par],
                                out_hbm.at[0, :, pl.ds(b0, _BW)], ssems[par])
          for par in range(4)]

    jj = lax.iota(jnp.int32, 16)

    def build_block(l, par):
        @plsc.parallel_loop(0, _G, 1, unroll=4)
        def g_body(g):
            base = idx_v[l, pl.ds(g * 16, 16)] * _EMBED
            bvec = jj + g * 16

            @plsc.parallel_loop(0, _EMBED, 16)
            def e_body(e0):
                base_e0 = base + e0
                for k in range(16):
                    rot = (jj + k) & 15
                    v = plsc.load_gather(table_v, [base_e0 + rot])
                    plsc.store_scatter(stage_v.at[par],
                                       [rot + e0, bvec], v)

    for par in range(4):
        build_block(par, par)
        fire_store(par, par)

    def body(j, carry):
        for par in range(4):
            l = 4 + j * 4 + par
            sd[par].wait()
            build_block(l, par)
            fire_store(l, par)
        return carry

    lax.fori_loop(0, (_L - 4) // 4, body, 0)

    for par in range(4):
        sd[par].wait()


def kernel(seqs, att_mask, word_embedding):
    del att_mask
    idx = seqs.T.reshape(_L, _NW, _BW).transpose(1, 0, 2)
    table = word_embedding.reshape(_VOCAB * _EMBED)
    out_t = _gather_kernel(idx, table)
    return out_t.transpose(2, 0, 1)


# speedup vs baseline: 1.7743x; 1.3791x over previous
import functools

import jax
import jax.numpy as jnp
from jax import lax
from jax.experimental import pallas as pl
from jax.experimental.pallas import tpu as pltpu
from jax.experimental.pallas import tpu_sc as plsc

_VOCAB = 1000
_EMBED = 64
_B = 4096
_L = 200

_NC = 2
_NS = 16
_NW = _NC * _NS
_BW = _B // _NW
_G = _BW // 16


@functools.partial(
    pl.kernel,
    mesh=plsc.VectorSubcoreMesh(core_axis_name="c", subcore_axis_name="s"),
    out_type=jax.ShapeDtypeStruct((_L, _EMBED, _B), jnp.float32),
    scratch_types=[
        pltpu.VMEM((_L, _BW), jnp.int32),
        pltpu.VMEM((_VOCAB * _EMBED,), jnp.float32),
        pltpu.VMEM((2, _EMBED, _BW), jnp.float32),
        [pltpu.SemaphoreType.DMA] * 2,
    ],
    compiler_params=pltpu.CompilerParams(use_tc_tiling_on_sc=True,
                                         needs_layout_passes=False),
)
def _gather_kernel(idx_hbm, table_hbm, out_hbm, idx_v, table_v, stage_v,
                   ssems):
    wid = lax.axis_index("s") * _NC + lax.axis_index("c")
    b0 = wid * _BW

    pltpu.sync_copy(table_hbm, table_v)
    pltpu.sync_copy(idx_hbm.at[wid], idx_v)

    def fire_store(l, par):
        return pltpu.async_copy(stage_v.at[par],
                                out_hbm.at[l, :, pl.ds(b0, _BW)], ssems[par])

    sd = [pltpu.make_async_copy(stage_v.at[par],
                                out_hbm.at[0, :, pl.ds(b0, _BW)], ssems[par])
          for par in (0, 1)]

    jj = lax.iota(jnp.int32, 16)

    def build_block(l, par):
        @plsc.parallel_loop(0, _G, 1, unroll=4)
        def g_body(g):
            base = idx_v[l, pl.ds(g * 16, 16)] * _EMBED
            bvec = jj + g * 16

            @plsc.parallel_loop(0, _EMBED, 16)
            def e_body(e0):
                for k in range(16):
                    re = (jj ^ k) + e0
                    v = plsc.load_gather(table_v, [base + re])
                    plsc.store_scatter(stage_v.at[par], [re, bvec], v)

    for par in (0, 1):
        build_block(par, par)
        fire_store(par, par)

    def body(j, carry):
        for par in (0, 1):
            l = 2 + j * 2 + par
            sd[par].wait()
            build_block(l, par)
            fire_store(l, par)
        return carry

    lax.fori_loop(0, (_L - 2) // 2, body, 0)

    sd[0].wait()
    sd[1].wait()


def kernel(seqs, att_mask, word_embedding):
    del att_mask
    idx = seqs.T.reshape(_L, _NW, _BW).transpose(1, 0, 2)
    table = word_embedding.reshape(_VOCAB * _EMBED)
    out_t = _gather_kernel(idx, table)
    return out_t.transpose(2, 0, 1)


# speedup vs baseline: 1.8578x; 1.0471x over previous
import functools

import jax
import jax.numpy as jnp
from jax import lax
from jax.experimental import pallas as pl
from jax.experimental.pallas import tpu as pltpu
from jax.experimental.pallas import tpu_sc as plsc

_VOCAB = 1000
_EMBED = 64
_B = 4096
_L = 200

_NC = 2
_NS = 16
_NW = _NC * _NS
_BW = _B // _NW
_G = _BW // 16


@functools.partial(
    pl.kernel,
    mesh=plsc.VectorSubcoreMesh(core_axis_name="c", subcore_axis_name="s"),
    out_type=jax.ShapeDtypeStruct((_L, _EMBED, _B), jnp.float32),
    scratch_types=[
        pltpu.VMEM((_L, _BW), jnp.int32),
        pltpu.VMEM((_VOCAB * _EMBED,), jnp.float32),
        pltpu.VMEM((2, _EMBED, _BW), jnp.float32),
        [pltpu.SemaphoreType.DMA] * 2,
    ],
    compiler_params=pltpu.CompilerParams(use_tc_tiling_on_sc=True,
                                         needs_layout_passes=False),
)
def _gather_kernel(idx_hbm, table_hbm, out_hbm, idx_v, table_v, stage_v,
                   ssems):
    wid = lax.axis_index("s") * _NC + lax.axis_index("c")
    b0 = wid * _BW

    pltpu.sync_copy(table_hbm, table_v)
    pltpu.sync_copy(idx_hbm.at[wid], idx_v)

    def fire_store(l, par):
        return pltpu.async_copy(stage_v.at[par],
                                out_hbm.at[l, :, pl.ds(b0, _BW)], ssems[par])

    sd = [pltpu.make_async_copy(stage_v.at[par],
                                out_hbm.at[0, :, pl.ds(b0, _BW)], ssems[par])
          for par in (0, 1)]

    jj = lax.iota(jnp.int32, 16)

    def build_block(l, par):
        @plsc.parallel_loop(0, _G, 1, unroll=4)
        def g_body(g):
            base = idx_v[l, pl.ds(g * 16, 16)] * _EMBED
            bvec = jj + g * 16

            @plsc.parallel_loop(0, _EMBED, 16)
            def e_body(e0):
                base_e0 = base + e0
                for k in range(16):
                    rot = (jj + k) & 15
                    v = plsc.load_gather(table_v, [base_e0 + rot])
                    plsc.store_scatter(stage_v.at[par],
                                       [rot + e0, bvec], v)

    for par in (0, 1):
        build_block(par, par)
        fire_store(par, par)

    def body(j, carry):
        for par in (0, 1):
            l = 2 + j * 2 + par
            sd[par].wait()
            build_block(l, par)
            fire_store(l, par)
        return carry

    lax.fori_loop(0, (_L - 2) // 2, body, 0)

    sd[0].wait()
    sd[1].wait()


def kernel(seqs, att_mask, word_embedding):
    del att_mask
    idx = seqs.T.reshape(_L, _NW, _BW).transpose(1, 0, 2)
    table = word_embedding.reshape(_VOCAB * _EMBED)
    out_t = _gather_kernel(idx, table)
    return out_t.transpose(2, 0, 1)
